# baseline (speedup 1.0000x reference)
import functools

import jax
import jax.numpy as jnp
from jax.experimental import pallas as pl
from jax.experimental.pallas import tpu as pltpu

N = 4096
H = 64
W = 64
FX = 64.0
FY = 64.0
NEAR = 0.3

P_TILE = 1024
K = 512
NCHUNK = N // K


def _quat_rotmat(q):
    w = q[..., 0]; x = q[..., 1]; y = q[..., 2]; z = q[..., 3]
    r = jnp.stack([
        1.0 - 2.0 * (y * y + z * z), 2.0 * (x * y - w * z), 2.0 * (x * z + w * y),
        2.0 * (x * y + w * z), 1.0 - 2.0 * (x * x + z * z), 2.0 * (y * z - w * x),
        2.0 * (x * z - w * y), 2.0 * (y * z + w * x), 1.0 - 2.0 * (x * x + y * y)
    ], axis=-1)
    return r.reshape(q.shape[:-1] + (3, 3))


def _prologue_body(rcw_ref, xin_ref, order_ref, rgb8_ref, gs_ref, rgbs_ref,
                   g_scr):
    j = pl.program_id(0)

    @pl.when(j == 0)
    def _compute_params():
        x = xin_ref[0:1, :]
        y = xin_ref[1:2, :]
        z = xin_ref[2:3, :]
        qw = xin_ref[3:4, :]
        qx = xin_ref[4:5, :]
        qy = xin_ref[5:6, :]
        qz = xin_ref[6:7, :]
        s0 = jax.nn.sigmoid(xin_ref[7:8, :])
        s1 = jax.nn.sigmoid(xin_ref[8:9, :])
        s2 = jax.nn.sigmoid(xin_ref[9:10, :])
        opa = jax.nn.sigmoid(xin_ref[10:11, :])

        zs = jnp.maximum(z, 1e-6)
        xr = x / zs
        yr = y / zs
        thx = W * 1.2 / (2.0 * FX)
        thy = H * 1.2 / (2.0 * FY)
        vis = ((z > NEAR) & (jnp.abs(xr) < thx) & (jnp.abs(yr) < thy))
        visf = vis.astype(jnp.float32)

        qn = jax.lax.rsqrt(qw * qw + qx * qx + qy * qy + qz * qz)
        w_ = qw * qn; x_ = qx * qn; y_ = qy * qn; z_ = qz * qn
        r00 = 1.0 - 2.0 * (y_ * y_ + z_ * z_)
        r01 = 2.0 * (x_ * y_ - w_ * z_)
        r02 = 2.0 * (x_ * z_ + w_ * y_)
        r10 = 2.0 * (x_ * y_ + w_ * z_)
        r11 = 1.0 - 2.0 * (x_ * x_ + z_ * z_)
        r12 = 2.0 * (y_ * z_ - w_ * x_)
        r20 = 2.0 * (x_ * z_ - w_ * y_)
        r21 = 2.0 * (y_ * z_ + w_ * x_)
        r22 = 1.0 - 2.0 * (x_ * x_ + y_ * y_)

        m00 = r00 * s0; m01 = r01 * s1; m02 = r02 * s2
        m10 = r10 * s0; m11 = r11 * s1; m12 = r12 * s2
        m20 = r20 * s0; m21 = r21 * s1; m22 = r22 * s2

        c3 = [[None] * 3 for _ in range(3)]
        mrows = [[m00, m01, m02], [m10, m11, m12], [m20, m21, m22]]
        for a_ in range(3):
            for b_ in range(a_, 3):
                c3[a_][b_] = (mrows[a_][0] * mrows[b_][0]
                              + mrows[a_][1] * mrows[b_][1]
                              + mrows[a_][2] * mrows[b_][2])
                c3[b_][a_] = c3[a_][b_]

        rcw = [[rcw_ref[0, 3 * a_ + b_] for b_ in range(3)] for a_ in range(3)]
        t = [[None] * 3 for _ in range(3)]
        for jj in range(3):
            for ll in range(3):
                t[jj][ll] = (c3[jj][0] * rcw[ll][0] + c3[jj][1] * rcw[ll][1]
                             + c3[jj][2] * rcw[ll][2])
        cc = [[None] * 3 for _ in range(3)]
        for ii in range(3):
            for ll in range(3):
                cc[ii][ll] = (rcw[ii][0] * t[0][ll] + rcw[ii][1] * t[1][ll]
                              + rcw[ii][2] * t[2][ll])

        fxz = FX / zs
        fyz = FY / zs
        jx = -FX * x / (zs * zs)
        jy = -FY * y / (zs * zs)
        a = fxz * fxz * cc[0][0] + 2.0 * fxz * jx * cc[0][2] + jx * jx * cc[2][2] + 0.3
        b = (fxz * fyz * cc[0][1] + fxz * jy * cc[0][2] + jx * fyz * cc[1][2]
             + jx * jy * cc[2][2])
        c = fyz * fyz * cc[1][1] + 2.0 * fyz * jy * cc[1][2] + jy * jy * cc[2][2] + 0.3

        det = jnp.maximum(a * c - b * b, 1e-8)
        dinv = 1.0 / det
        i00 = jnp.where(vis, c * dinv, 0.0)
        i01 = jnp.where(vis, -b * dinv, 0.0)
        i11 = jnp.where(vis, a * dinv, 0.0)
        mux = jnp.where(vis, FX * xr + W / 2.0, 0.0)
        muy = jnp.where(vis, FY * yr + H / 2.0, 0.0)
        opav = opa * visf

        g_scr[0:1, :] = mux
        g_scr[1:2, :] = muy
        g_scr[2:3, :] = i00
        g_scr[3:4, :] = i01
        g_scr[4:5, :] = i11
        g_scr[5:6, :] = opav
        g_scr[6:7, :] = jnp.zeros_like(mux)
        g_scr[7:8, :] = jnp.zeros_like(mux)

    order_row = order_ref[pl.ds(j, 1), :]
    oh = (jax.lax.broadcasted_iota(jnp.int32, (N, K), 0)
          == order_row).astype(jnp.float32)
    gs_ref[...] = jax.lax.dot_general(
        g_scr[...], oh, (((1,), (0,)), ((), ())),
        preferred_element_type=jnp.float32,
        precision=jax.lax.Precision.HIGHEST)
    rgbs_ref[...] = jax.lax.dot_general(
        oh, rgb8_ref[...], (((0,), (0,)), ((), ())),
        preferred_element_type=jnp.float32,
        precision=jax.lax.Precision.HIGHEST)


def _splat_body(g_ref, rgb_ref, out_ref, carry_ref):
    i = pl.program_id(0)
    j = pl.program_id(1)

    @pl.when(j == 0)
    def _init():
        carry_ref[...] = jnp.ones_like(carry_ref)
        out_ref[...] = jnp.zeros_like(out_ref)

    mux = g_ref[0:1, :]
    muy = g_ref[1:2, :]
    i00 = g_ref[2:3, :]
    i01 = g_ref[3:4, :]
    i11 = g_ref[4:5, :]
    opav = g_ref[5:6, :]

    row = i * P_TILE + jax.lax.broadcasted_iota(jnp.int32, (P_TILE, 1), 0)
    pxx = (row % W).astype(jnp.float32) + 0.5
    pyy = (row // W).astype(jnp.float32) + 0.5

    dx = pxx - mux
    dy = pyy - muy
    power = -0.5 * (i00 * dx * dx + 2.0 * i01 * dx * dy + i11 * dy * dy)
    alpha = jnp.minimum(opav * jnp.exp(power), 0.999)
    u = 1.0 - alpha + 1e-10

    c = u
    s = 1
    while s < K:
        shifted = jnp.concatenate(
            [jnp.ones((P_TILE, s), jnp.float32), c[:, :K - s]], axis=1)
        c = c * shifted
        s *= 2
    c_excl = jnp.concatenate(
        [jnp.ones((P_TILE, 1), jnp.float32), c[:, :K - 1]], axis=1)

    t_prev = carry_ref[...] * c_excl
    wgt = t_prev * alpha
    out_ref[...] += jax.lax.dot_general(
        wgt, rgb_ref[...], (((1,), (0,)), ((), ())),
        preferred_element_type=jnp.float32)
    carry_ref[...] = carry_ref[...] * c[:, K - 1:K]


@functools.partial(jax.jit)
def kernel(pos, rgb, opacity, quaternion, scale, qvec, tvec):
    f32 = jnp.float32
    Rcw = _quat_rotmat(qvec / jnp.linalg.norm(qvec))
    p_cam = pos @ Rcw.T + tvec
    z = p_cam[:, 2]
    order = jnp.argsort(z)

    xin = jnp.zeros((16, N), f32)
    xin = xin.at[0:3].set(p_cam.T)
    xin = xin.at[3:7].set(quaternion.T)
    xin = xin.at[7:10].set(scale.T)
    xin = xin.at[10].set(opacity)
    rgb8 = jnp.zeros((N, 8), f32).at[:, :3].set(rgb)
    order2d = order.astype(jnp.int32).reshape(NCHUNK, K)
    rcw_flat = jnp.reshape(Rcw, (1, 9))

    gs, rgbs = pl.pallas_call(
        _prologue_body,
        grid=(NCHUNK,),
        in_specs=[
            pl.BlockSpec(memory_space=pltpu.SMEM),
            pl.BlockSpec((16, N), lambda j: (0, 0)),
            pl.BlockSpec((NCHUNK, K), lambda j: (0, 0)),
            pl.BlockSpec((N, 8), lambda j: (0, 0)),
        ],
        out_specs=[
            pl.BlockSpec((8, K), lambda j: (0, j)),
            pl.BlockSpec((K, 8), lambda j: (j, 0)),
        ],
        out_shape=[
            jax.ShapeDtypeStruct((8, N), f32),
            jax.ShapeDtypeStruct((N, 8), f32),
        ],
        scratch_shapes=[pltpu.VMEM((8, N), f32)],
    )(rcw_flat, xin, order2d, rgb8)

    n_p = (H * W) // P_TILE
    out = pl.pallas_call(
        _splat_body,
        grid=(n_p, NCHUNK),
        in_specs=[
            pl.BlockSpec((8, K), lambda i, j: (0, j)),
            pl.BlockSpec((K, 8), lambda i, j: (j, 0)),
        ],
        out_specs=pl.BlockSpec((P_TILE, 8), lambda i, j: (i, 0)),
        out_shape=jax.ShapeDtypeStruct((H * W, 8), f32),
        scratch_shapes=[pltpu.VMEM((P_TILE, 1), f32)],
    )(gs, rgbs)
    return out[:, :3].reshape(H, W, 3)


# speedup vs baseline: 1.0537x; 1.0537x over previous
import functools

import jax
import jax.numpy as jnp
from jax.experimental import pallas as pl
from jax.experimental.pallas import tpu as pltpu

N = 4096
H = 64
W = 64
FX = 64.0
FY = 64.0
NEAR = 0.3

P_TILE = 1024
K = 512
NCHUNK = N // K


def _quat_rotmat(q):
    w = q[..., 0]; x = q[..., 1]; y = q[..., 2]; z = q[..., 3]
    r = jnp.stack([
        1.0 - 2.0 * (y * y + z * z), 2.0 * (x * y - w * z), 2.0 * (x * z + w * y),
        2.0 * (x * y + w * z), 1.0 - 2.0 * (x * x + z * z), 2.0 * (y * z - w * x),
        2.0 * (x * z - w * y), 2.0 * (y * z + w * x), 1.0 - 2.0 * (x * x + y * y)
    ], axis=-1)
    return r.reshape(q.shape[:-1] + (3, 3))


def _prologue_body(rcw_ref, xin_ref, order_ref, rgb8_ref, gs_ref, rgbs_ref,
                   gh_scr, gl_scr, rgbh_scr, rgbl_scr):
    j = pl.program_id(0)

    @pl.when(j == 0)
    def _compute_params():
        x = xin_ref[0:1, :]
        y = xin_ref[1:2, :]
        z = xin_ref[2:3, :]
        qw = xin_ref[3:4, :]
        qx = xin_ref[4:5, :]
        qy = xin_ref[5:6, :]
        qz = xin_ref[6:7, :]
        s0 = jax.nn.sigmoid(xin_ref[7:8, :])
        s1 = jax.nn.sigmoid(xin_ref[8:9, :])
        s2 = jax.nn.sigmoid(xin_ref[9:10, :])
        opa = jax.nn.sigmoid(xin_ref[10:11, :])

        zs = jnp.maximum(z, 1e-6)
        xr = x / zs
        yr = y / zs
        thx = W * 1.2 / (2.0 * FX)
        thy = H * 1.2 / (2.0 * FY)
        vis = ((z > NEAR) & (jnp.abs(xr) < thx) & (jnp.abs(yr) < thy))
        visf = vis.astype(jnp.float32)

        qn = jax.lax.rsqrt(qw * qw + qx * qx + qy * qy + qz * qz)
        w_ = qw * qn; x_ = qx * qn; y_ = qy * qn; z_ = qz * qn
        r00 = 1.0 - 2.0 * (y_ * y_ + z_ * z_)
        r01 = 2.0 * (x_ * y_ - w_ * z_)
        r02 = 2.0 * (x_ * z_ + w_ * y_)
        r10 = 2.0 * (x_ * y_ + w_ * z_)
        r11 = 1.0 - 2.0 * (x_ * x_ + z_ * z_)
        r12 = 2.0 * (y_ * z_ - w_ * x_)
        r20 = 2.0 * (x_ * z_ - w_ * y_)
        r21 = 2.0 * (y_ * z_ + w_ * x_)
        r22 = 1.0 - 2.0 * (x_ * x_ + y_ * y_)

        m00 = r00 * s0; m01 = r01 * s1; m02 = r02 * s2
        m10 = r10 * s0; m11 = r11 * s1; m12 = r12 * s2
        m20 = r20 * s0; m21 = r21 * s1; m22 = r22 * s2

        c3 = [[None] * 3 for _ in range(3)]
        mrows = [[m00, m01, m02], [m10, m11, m12], [m20, m21, m22]]
        for a_ in range(3):
            for b_ in range(a_, 3):
                c3[a_][b_] = (mrows[a_][0] * mrows[b_][0]
                              + mrows[a_][1] * mrows[b_][1]
                              + mrows[a_][2] * mrows[b_][2])
                c3[b_][a_] = c3[a_][b_]

        rcw = [[rcw_ref[0, 3 * a_ + b_] for b_ in range(3)] for a_ in range(3)]
        t = [[None] * 3 for _ in range(3)]
        for jj in range(3):
            for ll in range(3):
                t[jj][ll] = (c3[jj][0] * rcw[ll][0] + c3[jj][1] * rcw[ll][1]
                             + c3[jj][2] * rcw[ll][2])
        cc = [[None] * 3 for _ in range(3)]
        for ii in range(3):
            for ll in range(3):
                cc[ii][ll] = (rcw[ii][0] * t[0][ll] + rcw[ii][1] * t[1][ll]
                              + rcw[ii][2] * t[2][ll])

        fxz = FX / zs
        fyz = FY / zs
        jx = -FX * x / (zs * zs)
        jy = -FY * y / (zs * zs)
        a = fxz * fxz * cc[0][0] + 2.0 * fxz * jx * cc[0][2] + jx * jx * cc[2][2] + 0.3
        b = (fxz * fyz * cc[0][1] + fxz * jy * cc[0][2] + jx * fyz * cc[1][2]
             + jx * jy * cc[2][2])
        c = fyz * fyz * cc[1][1] + 2.0 * fyz * jy * cc[1][2] + jy * jy * cc[2][2] + 0.3

        det = jnp.maximum(a * c - b * b, 1e-8)
        dinv = 1.0 / det
        h00 = jnp.where(vis, -0.5 * (c * dinv), 0.0)
        h01 = jnp.where(vis, b * dinv, 0.0)
        h11 = jnp.where(vis, -0.5 * (a * dinv), 0.0)
        mux = jnp.where(vis, FX * xr + W / 2.0, 0.0)
        muy = jnp.where(vis, FY * yr + H / 2.0, 0.0)
        opav = opa * visf

        grows = [mux, muy, h00, h01, h11, opav,
                 jnp.zeros_like(mux), jnp.zeros_like(mux)]
        g = jnp.concatenate(grows, axis=0)
        ghi = g.astype(jnp.bfloat16)
        gh_scr[...] = ghi
        gl_scr[...] = (g - ghi.astype(jnp.float32)).astype(jnp.bfloat16)
        rgbf = rgb8_ref[...]
        rgbh = rgbf.astype(jnp.bfloat16)
        rgbh_scr[...] = rgbh
        rgbl_scr[...] = (rgbf - rgbh.astype(jnp.float32)).astype(jnp.bfloat16)

    order_row = order_ref[pl.ds(j, 1), :]
    oh = (jax.lax.broadcasted_iota(jnp.int32, (N, K), 0)
          == order_row).astype(jnp.float32).astype(jnp.bfloat16)
    dn_nk = (((1,), (0,)), ((), ()))
    gs_ref[...] = (
        jax.lax.dot_general(gh_scr[...], oh, dn_nk,
                            preferred_element_type=jnp.float32)
        + jax.lax.dot_general(gl_scr[...], oh, dn_nk,
                              preferred_element_type=jnp.float32))
    dn_t = (((0,), (0,)), ((), ()))
    rgbs_ref[...] = (
        jax.lax.dot_general(oh, rgbh_scr[...], dn_t,
                            preferred_element_type=jnp.float32)
        + jax.lax.dot_general(oh, rgbl_scr[...], dn_t,
                              preferred_element_type=jnp.float32))


def _ones_shift(c, s):
    return jnp.concatenate(
        [jnp.ones((P_TILE, s), jnp.float32), c[:, :K - s]], axis=1)


def _splat_body(g_ref, rgb_ref, out_ref, carry_ref):
    i = pl.program_id(0)
    j = pl.program_id(1)

    @pl.when(j == 0)
    def _init():
        carry_ref[...] = jnp.ones_like(carry_ref)
        out_ref[...] = jnp.zeros_like(out_ref)

    mux = g_ref[0:1, :]
    muy = g_ref[1:2, :]
    h00 = g_ref[2:3, :]
    h01 = g_ref[3:4, :]
    h11 = g_ref[4:5, :]
    opav = g_ref[5:6, :]

    row = i * P_TILE + jax.lax.broadcasted_iota(jnp.int32, (P_TILE, 1), 0)
    pxx = (row % W).astype(jnp.float32) + 0.5
    pyy = (row // W).astype(jnp.float32) + 0.5

    dx = pxx - mux
    dy = pyy - muy
    power = dx * (h00 * dx + h01 * dy) + h11 * (dy * dy)
    alpha = jnp.minimum(opav * jnp.exp(power), 0.999)
    u = 1.0 - alpha

    c = u * _ones_shift(u, 1) * _ones_shift(u, 2) * _ones_shift(u, 3)
    c = c * _ones_shift(c, 4) * _ones_shift(c, 8) * _ones_shift(c, 12)
    c = c * _ones_shift(c, 16) * _ones_shift(c, 32) * _ones_shift(c, 48)
    c = c * _ones_shift(c, 64) * _ones_shift(c, 128) * _ones_shift(c, 192)
    c = c * _ones_shift(c, 256)
    c_excl = _ones_shift(c, 1)

    t_prev = carry_ref[...] * c_excl
    wgt = t_prev * alpha
    out_ref[...] += jax.lax.dot_general(
        wgt, rgb_ref[...], (((1,), (0,)), ((), ())),
        preferred_element_type=jnp.float32)
    carry_ref[...] = carry_ref[...] * c[:, K - 1:K]


@functools.partial(jax.jit)
def kernel(pos, rgb, opacity, quaternion, scale, qvec, tvec):
    f32 = jnp.float32
    Rcw = _quat_rotmat(qvec / jnp.linalg.norm(qvec))
    p_cam = pos @ Rcw.T + tvec
    z = p_cam[:, 2]
    order = jnp.argsort(z)

    xin = jnp.zeros((16, N), f32)
    xin = xin.at[0:3].set(p_cam.T)
    xin = xin.at[3:7].set(quaternion.T)
    xin = xin.at[7:10].set(scale.T)
    xin = xin.at[10].set(opacity)
    rgb8 = jnp.zeros((N, 8), f32).at[:, :3].set(rgb)
    order2d = order.astype(jnp.int32).reshape(NCHUNK, K)
    rcw_flat = jnp.reshape(Rcw, (1, 9))

    gs, rgbs = pl.pallas_call(
        _prologue_body,
        grid=(NCHUNK,),
        in_specs=[
            pl.BlockSpec(memory_space=pltpu.SMEM),
            pl.BlockSpec((16, N), lambda j: (0, 0)),
            pl.BlockSpec((NCHUNK, K), lambda j: (0, 0)),
            pl.BlockSpec((N, 8), lambda j: (0, 0)),
        ],
        out_specs=[
            pl.BlockSpec((8, K), lambda j: (0, j)),
            pl.BlockSpec((K, 8), lambda j: (j, 0)),
        ],
        out_shape=[
            jax.ShapeDtypeStruct((8, N), f32),
            jax.ShapeDtypeStruct((N, 8), f32),
        ],
        scratch_shapes=[
            pltpu.VMEM((8, N), jnp.bfloat16),
            pltpu.VMEM((8, N), jnp.bfloat16),
            pltpu.VMEM((N, 8), jnp.bfloat16),
            pltpu.VMEM((N, 8), jnp.bfloat16),
        ],
    )(rcw_flat, xin, order2d, rgb8)

    n_p = (H * W) // P_TILE
    out = pl.pallas_call(
        _splat_body,
        grid=(n_p, NCHUNK),
        in_specs=[
            pl.BlockSpec((8, K), lambda i, j: (0, j)),
            pl.BlockSpec((K, 8), lambda i, j: (j, 0)),
        ],
        out_specs=pl.BlockSpec((P_TILE, 8), lambda i, j: (i, 0)),
        out_shape=jax.ShapeDtypeStruct((H * W, 8), f32),
        scratch_shapes=[pltpu.VMEM((P_TILE, 1), f32)],
    )(gs, rgbs)
    return out[:, :3].reshape(H, W, 3)
